# baseline (speedup 1.0000x reference)
import functools

import jax
import jax.numpy as jnp
from jax import lax
from jax.experimental import pallas as pl
from jax.experimental.pallas import tpu as pltpu
from jax.experimental.pallas import tpu_sc as plsc

_NC = 2
_NS = 16
_LANES = 16
_NW = _NC * _NS


def _make_pair_dot(batch: int, dim: int):
    assert batch % _NW == 0
    b_per_w = batch // _NW
    chunk = 128
    n_chunks = b_per_w // chunk
    groups_per_chunk = chunk // _LANES

    mesh = plsc.VectorSubcoreMesh(core_axis_name="c", subcore_axis_name="s")

    @functools.partial(
        pl.kernel,
        out_type=jax.ShapeDtypeStruct((batch,), jnp.float32),
        mesh=mesh,
        compiler_params=pltpu.CompilerParams(needs_layout_passes=False),
        scratch_types=[
            pltpu.VMEM((b_per_w,), jnp.int32),
            pltpu.VMEM((b_per_w,), jnp.int32),
            pltpu.VMEM((b_per_w,), jnp.int32),
            pltpu.VMEM((b_per_w,), jnp.int32),
            pltpu.VMEM((chunk, 2 * dim), jnp.float32),
            pltpu.VMEM((chunk, 2 * dim), jnp.float32),
            pltpu.VMEM((b_per_w,), jnp.float32),
            pltpu.SemaphoreType.DMA,
        ],
    )
    def pair_dot(i_hbm, j_hbm, view_hbm, out_hbm,
                 idx_i, idx_j, row_i, row_j, buf_i, buf_j, res, sem):
        wid = lax.axis_index("s") * _NC + lax.axis_index("c")
        base = wid * b_per_w

        pltpu.sync_copy(i_hbm.at[pl.ds(base, b_per_w)], idx_i)
        pltpu.sync_copy(j_hbm.at[pl.ds(base, b_per_w)], idx_j)

        lanes = lax.iota(jnp.int32, _LANES)

        def shift_body(v, _):
            sl = pl.ds(v * _LANES, _LANES)
            row_i[sl] = lax.shift_right_logical(idx_i[sl], 1)
            row_j[sl] = lax.shift_right_logical(idx_j[sl], 1)
            return _
        lax.fori_loop(0, b_per_w // _LANES, shift_body, 0)

        def chunk_body(c, _):
            sl = pl.ds(c * chunk, chunk)
            cp_i = pltpu.async_copy(view_hbm.at[row_i.at[sl]], buf_i, sem)
            cp_j = pltpu.async_copy(view_hbm.at[row_j.at[sl]], buf_j, sem)
            cp_i.wait()
            cp_j.wait()

            def group_body(g, carry):
                p0 = c * chunk + g * _LANES
                rows = g * _LANES + lanes
                col_i = (idx_i[pl.ds(p0, _LANES)] & 1) * dim
                col_j = (idx_j[pl.ds(p0, _LANES)] & 1) * dim
                acc = jnp.zeros((_LANES,), jnp.float32)
                for d in range(dim):
                    vi = plsc.load_gather(buf_i, [rows, col_i + d])
                    vj = plsc.load_gather(buf_j, [rows, col_j + d])
                    acc = acc + vi * vj
                res[pl.ds(p0, _LANES)] = acc
                return carry

            lax.fori_loop(0, groups_per_chunk, group_body, 0)
            return _

        lax.fori_loop(0, n_chunks, chunk_body, 0)

        pltpu.sync_copy(res, out_hbm.at[pl.ds(base, b_per_w)])

    return pair_dot


def kernel(pair, table):
    batch = pair.shape[0]
    nrows, dim = table.shape
    i = pair[:, 0].astype(jnp.int32)
    j = pair[:, 1].astype(jnp.int32)
    view = table.reshape(nrows // 2, 2 * dim)
    sim = _make_pair_dot(batch, dim)(i, j, view)
    return sim[:, None]


# speedup vs baseline: 1.6668x; 1.6668x over previous
import functools

import jax
import jax.numpy as jnp
from jax import lax
from jax.experimental import pallas as pl
from jax.experimental.pallas import tpu as pltpu
from jax.experimental.pallas import tpu_sc as plsc

_NC = 2
_NS = 16
_LANES = 16
_NW = _NC * _NS


def _make_extract(batch, dim, nrows):
    nrefs = batch * 2
    nblk = (nrows + 127) // 128
    cap = nrefs + 2 * _LANES
    scan_chunk = 2048
    dummy0 = nrefs

    mesh = plsc.VectorSubcoreMesh(core_axis_name="c", subcore_axis_name="s")

    @functools.partial(
        pl.kernel,
        out_type=jax.ShapeDtypeStruct((nrefs + _LANES, 128), jnp.float32),
        mesh=mesh,
        compiler_params=pltpu.CompilerParams(
            needs_layout_passes=False, disable_bounds_checks=True),
        scratch_types=[
            pltpu.VMEM((scan_chunk,), jnp.int32),
            pltpu.VMEM((cap,), jnp.int32),
            pltpu.VMEM((cap,), jnp.int32),
            pltpu.VMEM((64, 128), jnp.float32),
            pltpu.VMEM((64, 128), jnp.float32),
            pltpu.VMEM((128, 128), jnp.float32),
            pltpu.VMEM((1, 128), jnp.int32),
            pltpu.SMEM((256,), jnp.int32),
            pltpu.SMEM((256,), jnp.int32),
            pltpu.SMEM((256,), jnp.int32),
            pltpu.SMEM((8,), jnp.int32),
            pltpu.SemaphoreType.DMA,
            pltpu.SemaphoreType.DMA,
        ],
    )
    def extract(ridx_hbm, tab_t, s_hbm,
                idxc, loc, srt, blka, blkb, bat, rowid,
                cnts, starts, cur, misc, semf, sems):
        wid = lax.axis_index("s") * _NC + lax.axis_index("c")
        lanes = lax.iota(jnp.int32, _LANES)
        lo = (wid * nblk) // _NW
        hi = ((wid + 1) * nblk) // _NW
        nb = hi - lo

        misc[0] = 0

        def scan_outer(c, _):
            pltpu.sync_copy(ridx_hbm.at[pl.ds(c * scan_chunk, scan_chunk)],
                            idxc)

            def scan_inner(g, _):
                v = idxc[pl.ds(g * _LANES, _LANES)]
                b = lax.shift_right_logical(v, 7)
                m = (b >= lo) & (b < hi)
                r2 = c * scan_chunk + g * _LANES + lanes
                ent = ((b - lo) << 22) | ((v & 127) << 15) | r2
                cnt = misc[0]
                plsc.store_compressed(loc.at[pl.ds(cnt, _LANES)], ent, mask=m)
                misc[0] = cnt + jnp.sum(m.astype(jnp.int32))
                return _

            lax.fori_loop(0, scan_chunk // _LANES, scan_inner, 0)
            return _

        lax.fori_loop(0, nrefs // scan_chunk, scan_outer, 0)
        cnt = misc[0]

        def zero_body(k, _):
            cnts[k] = 0
            return _
        lax.fori_loop(0, 256, zero_body, 0)

        def hist_body(t, carry):
            ev = plsc.load_gather(loc, [jnp.full((_LANES,), t, jnp.int32)])
            e = jnp.max(ev)
            bl = lax.shift_right_logical(e, 22)
            cnts[bl] = cnts[bl] + 1
            return carry

        lax.fori_loop(0, cnt, hist_body, 0)

        misc[2] = 0

        def prefix_body(k, _):
            s = misc[2]
            starts[k] = s
            cur[k] = s
            misc[2] = s + cnts[k]
            return _

        lax.fori_loop(0, 256, prefix_body, 0)

        def place_body(t, carry):
            ev = plsc.load_gather(loc, [jnp.full((_LANES,), t, jnp.int32)])
            e = jnp.max(ev)
            bl = lax.shift_right_logical(e, 22)
            pos = cur[bl]
            cur[bl] = pos + 1
            posv = jnp.full((_LANES,), pos, jnp.int32)
            payv = jnp.full((_LANES,), e & 0x3FFFFF, jnp.int32)
            plsc.store_scatter(srt, [posv], payv, mask=lanes == 0)
            return carry

        lax.fori_loop(0, cnt, place_body, 0)

        misc[1] = 0

        def reset_rowid():
            def rbody(k, _):
                rowid[0, pl.ds(k * _LANES, _LANES)] = dummy0 + lanes
                return _
            lax.fori_loop(0, 128 // _LANES, rbody, 0)

        reset_rowid()

        def flush():
            cp = pltpu.async_copy(bat, s_hbm.at[rowid.at[0]], sems)
            cp.wait()
            misc[1] = 0
            reset_rowid()

        def issue(b, blk):
            return pltpu.async_copy(
                tab_t.at[:, pl.ds((lo + b) * 128, 128)], blk, semf)

        def wait_fetch():
            pltpu.make_async_copy(
                tab_t.at[:, pl.ds(0, 128)], blka, semf).wait()

        def process(b, blk):
            start = starts[b]
            cb = cnts[b]
            qlo = start // _LANES
            qhi = (start + cb + _LANES - 1) // _LANES

            def gbody(q, carry):
                base = q * _LANES
                pos = base + lanes
                ev = srt[pl.ds(base, _LANES)]
                win = (pos >= start) & (pos < start + cb)
                colv = lax.shift_right_logical(ev, 15) & 127
                slotv = ev & 0x7FFF
                rb = misc[1]
                rank = jnp.cumsum(win.astype(jnp.int32))
                rtgt = rb + rank - 1

                for d in range(64):
                    vals = plsc.load_gather(
                        blk, [jnp.full((_LANES,), d, jnp.int32), colv])
                    plsc.store_scatter(
                        bat, [rtgt, jnp.full((_LANES,), d, jnp.int32)],
                        vals, mask=win)

                plsc.store_compressed(rowid.at[0, pl.ds(rb, _LANES)], slotv,
                                      mask=win)
                misc[1] = rb + jnp.sum(win.astype(jnp.int32))

                @pl.when(misc[1] >= 112)
                def _flush_if_full():
                    flush()

                return carry

            lax.fori_loop(qlo, qhi, gbody, 0)

        @pl.when(nb > 0)
        def _():
            issue(0, blka)
            wait_fetch()

        def sweep_outer(t, carry):
            b0 = 2 * t

            @pl.when(b0 < nb)
            def _phase_a():
                @pl.when(b0 + 1 < nb)
                def _issue_next():
                    issue(b0 + 1, blkb)
                process(b0, blka)

                @pl.when(b0 + 1 < nb)
                def _wait_next():
                    wait_fetch()

            b1 = b0 + 1

            @pl.when(b1 < nb)
            def _phase_b():
                @pl.when(b1 + 1 < nb)
                def _issue_next():
                    issue(b1 + 1, blka)
                process(b1, blkb)

                @pl.when(b1 + 1 < nb)
                def _wait_next():
                    wait_fetch()

            return carry

        lax.fori_loop(0, (nb + 1) // 2, sweep_outer, 0)

        @pl.when(misc[1] > 0)
        def _():
            flush()

    return extract


def _make_dot(batch):
    nrefs = batch * 2
    b_per_w = batch // _NW
    chunk = 128
    n_chunks = b_per_w // chunk

    mesh = plsc.VectorSubcoreMesh(core_axis_name="c", subcore_axis_name="s")

    @functools.partial(
        pl.kernel,
        out_type=jax.ShapeDtypeStruct((batch,), jnp.float32),
        mesh=mesh,
        compiler_params=pltpu.CompilerParams(needs_layout_passes=False),
        scratch_types=[
            pltpu.VMEM((2 * chunk, 128), jnp.float32),
            pltpu.VMEM((2 * chunk, 128), jnp.float32),
            pltpu.VMEM((b_per_w,), jnp.float32),
            pltpu.SemaphoreType.DMA,
        ],
    )
    def dot(s_hbm, out_hbm, bufa, bufb, res, sem):
        wid = lax.axis_index("s") * _NC + lax.axis_index("c")
        base = wid * b_per_w
        lanes = lax.iota(jnp.int32, _LANES)

        def issue(c, buf):
            return pltpu.async_copy(
                s_hbm.at[pl.ds((base + c * chunk) * 2, 2 * chunk)], buf, sem)

        def wait_fetch():
            pltpu.make_async_copy(
                s_hbm.at[pl.ds(0, 2 * chunk)], bufa, sem).wait()

        def process(c, buf):
            def gbody(g, _):
                rows_i = (g * _LANES + lanes) * 2
                rows_j = rows_i + 1
                acc = jnp.zeros((_LANES,), jnp.float32)
                for d in range(64):
                    dv = jnp.full((_LANES,), d, jnp.int32)
                    vi = plsc.load_gather(buf, [rows_i, dv])
                    vj = plsc.load_gather(buf, [rows_j, dv])
                    acc = acc + vi * vj
                res[pl.ds(c * chunk + g * _LANES, _LANES)] = acc
                return _

            lax.fori_loop(0, chunk // _LANES, gbody, 0)

        issue(0, bufa)
        wait_fetch()
        bufs = [bufa, bufb]
        for c in range(n_chunks):
            if c + 1 < n_chunks:
                issue(c + 1, bufs[(c + 1) % 2])
            process(c, bufs[c % 2])
            if c + 1 < n_chunks:
                wait_fetch()

        pltpu.sync_copy(res, out_hbm.at[pl.ds(base, b_per_w)])

    return dot


def kernel(pair, table):
    batch = pair.shape[0]
    nrows, dim = table.shape
    ridx = pair.astype(jnp.int32).reshape(batch * 2)
    staged = _make_extract(batch, dim, nrows)(ridx, table.T)
    sim = _make_dot(batch)(staged)
    return sim[:, None]
